# Initial kernel scaffold; baseline (speedup 1.0000x reference)
import jax
import jax.numpy as jnp
from jax.experimental import pallas as pl


def kernel(mh_out, W_ln, b_ln, W_noise, b_noise, noise):
    raise NotImplementedError("write your pallas kernel here")


# speedup vs baseline: 2.9234x; 2.9234x over previous
import functools

import jax
import jax.numpy as jnp
from jax.experimental import pallas as pl
from jax.experimental.pallas import tpu as pltpu

N_TOK = 8192
N_EMBD = 768
NUM_EXP = 16
BLK = 1024


def _router_body(x_ref, wl_ref, bl_ref, wn_ref, bn_ref, nz_ref,
                 ro_ref, ind_ref):
    x = x_ref[...]
    logits = jnp.dot(x, wl_ref[...], preferred_element_type=jnp.float32)
    logits = logits + bl_ref[...][None, :]
    nl = jnp.dot(x, wn_ref[...], preferred_element_type=jnp.float32)
    nl = nl + bn_ref[...][None, :]
    sp = jnp.maximum(nl, 0.0) + jnp.log1p(jnp.exp(-jnp.abs(nl)))
    noisy = logits + nz_ref[...] * sp

    col = jax.lax.broadcasted_iota(jnp.int32, noisy.shape, 1)
    m1 = jnp.max(noisy, axis=-1, keepdims=True)
    i1 = jnp.min(jnp.where(noisy == m1, col, NUM_EXP), axis=-1, keepdims=True)
    hit1 = col == i1
    masked = jnp.where(hit1, -jnp.inf, noisy)
    m2 = jnp.max(masked, axis=-1, keepdims=True)
    i2 = jnp.min(jnp.where(masked == m2, col, NUM_EXP), axis=-1, keepdims=True)
    hit2 = col == i2

    e2 = jnp.exp(m2 - m1)
    denom = 1.0 + e2
    p1 = 1.0 / denom
    p2 = e2 / denom
    ro_ref[...] = jnp.where(hit1, p1, jnp.where(hit2, p2, 0.0))
    ind_ref[...] = jnp.concatenate([i1, i2], axis=-1)


@jax.jit
def kernel(mh_out, W_ln, b_ln, W_noise, b_noise, noise):
    grid = (N_TOK // BLK,)
    ro, ind = pl.pallas_call(
        _router_body,
        grid=grid,
        in_specs=[
            pl.BlockSpec((BLK, N_EMBD), lambda i: (i, 0)),
            pl.BlockSpec((N_EMBD, NUM_EXP), lambda i: (0, 0)),
            pl.BlockSpec((NUM_EXP,), lambda i: (0,)),
            pl.BlockSpec((N_EMBD, NUM_EXP), lambda i: (0, 0)),
            pl.BlockSpec((NUM_EXP,), lambda i: (0,)),
            pl.BlockSpec((BLK, NUM_EXP), lambda i: (i, 0)),
        ],
        out_specs=[
            pl.BlockSpec((BLK, NUM_EXP), lambda i: (i, 0)),
            pl.BlockSpec((BLK, 2), lambda i: (i, 0)),
        ],
        out_shape=[
            jax.ShapeDtypeStruct((N_TOK, NUM_EXP), jnp.float32),
            jax.ShapeDtypeStruct((N_TOK, 2), jnp.int32),
        ],
    )(mh_out, W_ln, b_ln, W_noise, b_noise, noise)
    return ro, ind
